# baseline (speedup 1.0000x reference)
import jax
import jax.numpy as jnp
from jax import lax
from jax.experimental import pallas as pl
from jax.experimental.pallas import tpu as pltpu
from jax.experimental.pallas import tpu_sc as plsc

_L = 16
_CA = 16
_D = 512
_R = 3 * _D
_CW = _CA * _R
_QA = 32
_NSC = 8


def _sc_body(x_hbm, der_hbm, out_hbm, x_v, der_v0, der_v1, out_v, red_v,
             sem0, sem1):
    wid = lax.axis_index("c") * 16 + lax.axis_index("s")
    q = wid // 4
    r = wid % 4
    lane = lax.broadcasted_iota(jnp.int32, (_L,), 0)
    lrow = lane * _R

    pltpu.sync_copy(x_hbm.at[pl.ds(q * _D, _D)], x_v)

    bufs = (der_v0, der_v1)
    sems = (sem0, sem1)
    zero = jnp.zeros((_L,), jnp.float32)

    for u in range(2):
        pltpu.async_copy(
            der_hbm.at[q, pl.ds(r * (_QA * _R) + u * _CW, _CW)], bufs[u],
            sems[u])

    sq_acc = zero
    for u in range(2):
        buf, sem = bufs[u], sems[u]
        pltpu.make_async_copy(der_hbm.at[q, pl.ds(0, _CW)], buf, sem).wait()

        def dloop(i, carry, buf=buf):
            a0, a1, a2, b0, b1, b2, i0, i1, i2 = carry
            xv = x_v[pl.ds(i * _L, _L)]
            for m in range(_L):
                g0 = plsc.load_gather(buf, [i0])
                g1 = plsc.load_gather(buf, [i1])
                g2 = plsc.load_gather(buf, [i2])
                i0, i1, i2 = i0 + 3, i1 + 3, i2 + 3
                xs = xv[m]
                if m % 2 == 0:
                    a0 = a0 + g0 * xs
                    a1 = a1 + g1 * xs
                    a2 = a2 + g2 * xs
                else:
                    b0 = b0 + g0 * xs
                    b1 = b1 + g1 * xs
                    b2 = b2 + g2 * xs
            return (a0, a1, a2, b0, b1, b2, i0, i1, i2)

        a0, a1, a2, b0, b1, b2, _, _, _ = lax.fori_loop(
            0, _D // _L, dloop, (zero,) * 6 + (lrow, lrow + 1, lrow + 2))
        a0, a1, a2 = a0 + b0, a1 + b1, a2 + b2
        sq_acc = sq_acc + a0 * a0 + a1 * a1 + a2 * a2

    for s in (8, 4, 2, 1):
        red_v[...] = sq_acc
        sq_acc = sq_acc + plsc.load_gather(red_v, [(lane + s) % _L])

    out_v[...] = sq_acc
    pltpu.sync_copy(out_v, out_hbm.at[wid])


def _tc_body(w3_ref, der_ref, out_ref):
    der = der_ref[0, 0].reshape(128, _R)
    y = lax.dot_general(der, w3_ref[0], (((1,), (1,)), ((), ())),
                        preferred_element_type=jnp.float32)
    out_ref[...] = jnp.sum(y * y, keepdims=True)[None]


def kernel(x, der_desc_wrt_coord):
    B, A, D, K = der_desc_wrt_coord.shape
    der2 = der_desc_wrt_coord.reshape(B, A * D * K)
    x_flat = x.reshape(B * D)

    sc = pl.kernel(
        _sc_body,
        out_type=jax.ShapeDtypeStruct((32, _L), jnp.float32),
        mesh=plsc.VectorSubcoreMesh(core_axis_name="c", subcore_axis_name="s"),
        compiler_params=pltpu.CompilerParams(needs_layout_passes=False),
        scratch_types=[
            pltpu.VMEM((_D,), jnp.float32),
            pltpu.VMEM((_CW,), jnp.float32),
            pltpu.VMEM((_CW,), jnp.float32),
            pltpu.VMEM((_L,), jnp.float32),
            pltpu.VMEM((_L,), jnp.float32),
            pltpu.SemaphoreType.DMA,
            pltpu.SemaphoreType.DMA,
        ],
    )
    out_sc = sc(x_flat, der2)

    j = jnp.arange(D * K, dtype=jnp.int32)
    sel = (j[None, :] % K) == jnp.arange(K, dtype=jnp.int32)[:, None]
    w3 = jnp.where(sel[None], x[:, None, j // K], 0.0)
    out_tc = pl.pallas_call(
        _tc_body,
        grid=(B - _NSC,),
        in_specs=[
            pl.BlockSpec((1, K, D * K), lambda b: (b + _NSC, 0, 0)),
            pl.BlockSpec((1, 1, A * D * K), lambda b: (b + _NSC, 0, 0)),
        ],
        out_specs=pl.BlockSpec((1, 1, 1), lambda b: (b, 0, 0)),
        out_shape=jax.ShapeDtypeStruct((B - _NSC, 1, 1), jnp.float32),
    )(w3, der2.reshape(B, 1, A * D * K))

    head = out_sc[:, 0].reshape(_NSC, 4).sum(axis=1)
    return jnp.concatenate([head, out_tc[:, 0, 0]])


# speedup vs baseline: 4.1713x; 4.1713x over previous
import jax
import jax.numpy as jnp
from jax import lax
from jax.experimental import pallas as pl
from jax.experimental.pallas import tpu as pltpu
from jax.experimental.pallas import tpu_sc as plsc

_L = 16
_CA = 16
_D = 512
_R = 3 * _D
_CW = _CA * _R
_QA = 32
_NSC = 8


def _sc_body(x_hbm, der_hbm, out_hbm, x_v, der_v0, der_v1, out_v, red_v,
             sem0, sem1):
    wid = lax.axis_index("c") * 16 + lax.axis_index("s")
    q = wid // 4
    r = wid % 4
    lane = lax.broadcasted_iota(jnp.int32, (_L,), 0)
    lrow = lane * _R

    pltpu.sync_copy(x_hbm.at[pl.ds(q * _D, _D)], x_v)

    bufs = (der_v0, der_v1)
    sems = (sem0, sem1)
    zero = jnp.zeros((_L,), jnp.float32)

    for u in range(2):
        pltpu.async_copy(
            der_hbm.at[q, pl.ds(r * _QA + u * _CA, _CA)], bufs[u], sems[u])

    c_init = (lane * 0, lane * 0 + 1, lane * 0 + 2)
    sq_acc = zero
    for u in range(2):
        buf, sem = bufs[u], sems[u]
        pltpu.make_async_copy(der_hbm.at[q, pl.ds(0, _CA)], buf, sem).wait()

        def dloop(i, carry, buf=buf):
            a0, a1, a2, b0, b1, b2, c0, c1, c2 = carry
            xv = x_v[pl.ds(i * _L, _L)]
            for m in range(_L):
                g0 = plsc.load_gather(buf, [lane, c0])
                g1 = plsc.load_gather(buf, [lane, c1])
                g2 = plsc.load_gather(buf, [lane, c2])
                c0, c1, c2 = c0 + 3, c1 + 3, c2 + 3
                xs = xv[m]
                if m % 2 == 0:
                    a0 = a0 + g0 * xs
                    a1 = a1 + g1 * xs
                    a2 = a2 + g2 * xs
                else:
                    b0 = b0 + g0 * xs
                    b1 = b1 + g1 * xs
                    b2 = b2 + g2 * xs
            return (a0, a1, a2, b0, b1, b2, c0, c1, c2)

        a0, a1, a2, b0, b1, b2, _, _, _ = lax.fori_loop(
            0, _D // _L, dloop, (zero,) * 6 + c_init)
        a0, a1, a2 = a0 + b0, a1 + b1, a2 + b2
        sq_acc = sq_acc + a0 * a0 + a1 * a1 + a2 * a2

    for s in (8, 4, 2, 1):
        red_v[...] = sq_acc
        sq_acc = sq_acc + plsc.load_gather(red_v, [(lane + s) % _L])

    out_v[...] = sq_acc
    pltpu.sync_copy(out_v, out_hbm.at[wid])


def _tc_body(w3_ref, der_ref, out_ref):
    y = lax.dot_general(der_ref[0], w3_ref[0], (((1,), (1,)), ((), ())),
                        preferred_element_type=jnp.float32)
    out_ref[...] = jnp.sum(y * y, keepdims=True)[None]


def kernel(x, der_desc_wrt_coord):
    B, A, D, K = der_desc_wrt_coord.shape
    der3 = der_desc_wrt_coord.reshape(B, A, D * K)
    x_flat = x.reshape(B * D)

    sc = pl.kernel(
        _sc_body,
        out_type=jax.ShapeDtypeStruct((32, _L), jnp.float32),
        mesh=plsc.VectorSubcoreMesh(core_axis_name="c", subcore_axis_name="s"),
        compiler_params=pltpu.CompilerParams(needs_layout_passes=False),
        scratch_types=[
            pltpu.VMEM((_D,), jnp.float32),
            pltpu.VMEM((_CA, _R), jnp.float32),
            pltpu.VMEM((_CA, _R), jnp.float32),
            pltpu.VMEM((_L,), jnp.float32),
            pltpu.VMEM((_L,), jnp.float32),
            pltpu.SemaphoreType.DMA,
            pltpu.SemaphoreType.DMA,
        ],
    )
    out_sc = sc(x_flat, der3)

    j = jnp.arange(D * K, dtype=jnp.int32)
    sel = (j[None, :] % K) == jnp.arange(K, dtype=jnp.int32)[:, None]
    w3 = jnp.where(sel[None], x[:, None, j // K], 0.0)
    out_tc = pl.pallas_call(
        _tc_body,
        grid=(B - _NSC,),
        in_specs=[
            pl.BlockSpec((1, K, D * K), lambda b: (b + _NSC, 0, 0)),
            pl.BlockSpec((1, A, D * K), lambda b: (b + _NSC, 0, 0)),
        ],
        out_specs=pl.BlockSpec((1, 1, 1), lambda b: (b, 0, 0)),
        out_shape=jax.ShapeDtypeStruct((B - _NSC, 1, 1), jnp.float32),
    )(w3, der3)

    head = out_sc[:, 0].reshape(_NSC, 4).sum(axis=1)
    return jnp.concatenate([head, out_tc[:, 0, 0]])


# speedup vs baseline: 4.6505x; 1.1149x over previous
import jax
import jax.numpy as jnp
from jax import lax
from jax.experimental import pallas as pl
from jax.experimental.pallas import tpu as pltpu
from jax.experimental.pallas import tpu_sc as plsc

_L = 16
_CA = 16
_D = 512
_R = 3 * _D
_CW = _CA * _R
_QA = 32
_NSC = 8


def _sc_body(x_hbm, der_hbm, out_hbm, x_v, der_v0, der_v1, out_v, red_v,
             sem0, sem1):
    wid = lax.axis_index("c") * 16 + lax.axis_index("s")
    q = wid // 4
    r = wid % 4
    lane = lax.broadcasted_iota(jnp.int32, (_L,), 0)
    lrow = lane * _R

    pltpu.sync_copy(x_hbm.at[pl.ds(q * _D, _D)], x_v)

    bufs = (der_v0, der_v1)
    sems = (sem0, sem1)
    zero = jnp.zeros((_L,), jnp.float32)

    for u in range(2):
        pltpu.async_copy(
            der_hbm.at[q, pl.ds(r * _QA + u * _CA, _CA)], bufs[u], sems[u])

    c_init = (lane * 0, lane * 0 + 1, lane * 0 + 2)
    sq_acc = zero
    for u in range(2):
        buf, sem = bufs[u], sems[u]
        pltpu.make_async_copy(der_hbm.at[q, pl.ds(0, _CA)], buf, sem).wait()

        def dloop(i, carry, buf=buf):
            a0, a1, a2, b0, b1, b2, c0, c1, c2 = carry
            xv = x_v[pl.ds(i * _L, _L)]
            for m in range(_L):
                g0 = plsc.load_gather(buf, [lane, c0])
                g1 = plsc.load_gather(buf, [lane, c1])
                g2 = plsc.load_gather(buf, [lane, c2])
                c0, c1, c2 = c0 + 3, c1 + 3, c2 + 3
                xs = xv[m]
                if m % 2 == 0:
                    a0 = a0 + g0 * xs
                    a1 = a1 + g1 * xs
                    a2 = a2 + g2 * xs
                else:
                    b0 = b0 + g0 * xs
                    b1 = b1 + g1 * xs
                    b2 = b2 + g2 * xs
            return (a0, a1, a2, b0, b1, b2, c0, c1, c2)

        a0, a1, a2, b0, b1, b2, _, _, _ = lax.fori_loop(
            0, _D // _L, dloop, (zero,) * 6 + c_init)
        a0, a1, a2 = a0 + b0, a1 + b1, a2 + b2
        sq_acc = sq_acc + a0 * a0 + a1 * a1 + a2 * a2

    for s in (8, 4, 2, 1):
        red_v[...] = sq_acc
        sq_acc = sq_acc + plsc.load_gather(red_v, [(lane + s) % _L])

    out_v[...] = sq_acc
    pltpu.sync_copy(out_v, out_hbm.at[wid])


def _tc_body(w3_ref, der_ref, out_ref):
    s = []
    for bb in range(2):
        y = lax.dot_general(der_ref[bb], w3_ref[bb], (((1,), (1,)), ((), ())),
                            preferred_element_type=jnp.float32)
        s.append(jnp.sum(y * y, keepdims=True))
    out_ref[...] = jnp.stack(s)


def kernel(x, der_desc_wrt_coord):
    B, A, D, K = der_desc_wrt_coord.shape
    der3 = der_desc_wrt_coord.reshape(B, A, D * K)
    x_flat = x.reshape(B * D)

    sc = pl.kernel(
        _sc_body,
        out_type=jax.ShapeDtypeStruct((32, _L), jnp.float32),
        mesh=plsc.VectorSubcoreMesh(core_axis_name="c", subcore_axis_name="s"),
        compiler_params=pltpu.CompilerParams(needs_layout_passes=False),
        scratch_types=[
            pltpu.VMEM((_D,), jnp.float32),
            pltpu.VMEM((_CA, _R), jnp.float32),
            pltpu.VMEM((_CA, _R), jnp.float32),
            pltpu.VMEM((_L,), jnp.float32),
            pltpu.VMEM((_L,), jnp.float32),
            pltpu.SemaphoreType.DMA,
            pltpu.SemaphoreType.DMA,
        ],
    )
    out_sc = sc(x_flat, der3)

    j = jnp.arange(D * K, dtype=jnp.int32)
    sel = (j[None, :] % K) == jnp.arange(K, dtype=jnp.int32)[:, None]
    w3 = jnp.where(sel[None], x[:, None, j // K], 0.0)
    out_tc = pl.pallas_call(
        _tc_body,
        grid=((B - _NSC) // 2,),
        in_specs=[
            pl.BlockSpec((2, K, D * K), lambda b: (b + _NSC // 2, 0, 0)),
            pl.BlockSpec((2, A, D * K), lambda b: (b + _NSC // 2, 0, 0)),
        ],
        out_specs=pl.BlockSpec((2, 1, 1), lambda b: (b, 0, 0)),
        out_shape=jax.ShapeDtypeStruct((B - _NSC, 1, 1), jnp.float32),
    )(w3, der3)

    head = out_sc[:, 0].reshape(_NSC, 4).sum(axis=1)
    return jnp.concatenate([head, out_tc[:, 0, 0]])


# speedup vs baseline: 4.8193x; 1.0363x over previous
import jax
import jax.numpy as jnp
from jax import lax
from jax.experimental import pallas as pl
from jax.experimental.pallas import tpu as pltpu
from jax.experimental.pallas import tpu_sc as plsc

_L = 16
_CA = 16
_D = 512
_R = 3 * _D
_CW = _CA * _R
_QA = 32
_NSC = 8


def _sc_body(x_hbm, der_hbm, out_hbm, x_v, der_v0, der_v1, out_v, red_v,
             sem0, sem1):
    wid = lax.axis_index("c") * 16 + lax.axis_index("s")
    q = wid // 4
    r = wid % 4
    lane = lax.broadcasted_iota(jnp.int32, (_L,), 0)
    lrow = lane * _R

    pltpu.sync_copy(x_hbm.at[pl.ds(q * _D, _D)], x_v)

    bufs = (der_v0, der_v1)
    sems = (sem0, sem1)
    zero = jnp.zeros((_L,), jnp.float32)

    for u in range(2):
        pltpu.async_copy(
            der_hbm.at[q, pl.ds(r * _QA + u * _CA, _CA)], bufs[u], sems[u])

    c_init = (lane * 0, lane * 0 + 1, lane * 0 + 2)
    sq_acc = zero
    for u in range(2):
        buf, sem = bufs[u], sems[u]
        pltpu.make_async_copy(der_hbm.at[q, pl.ds(0, _CA)], buf, sem).wait()

        def dloop(i, carry, buf=buf):
            a0, a1, a2, b0, b1, b2, c0, c1, c2 = carry
            xv = x_v[pl.ds(i * _L, _L)]
            for m in range(_L):
                g0 = plsc.load_gather(buf, [lane, c0])
                g1 = plsc.load_gather(buf, [lane, c1])
                g2 = plsc.load_gather(buf, [lane, c2])
                c0, c1, c2 = c0 + 3, c1 + 3, c2 + 3
                xs = xv[m]
                if m % 2 == 0:
                    a0 = a0 + g0 * xs
                    a1 = a1 + g1 * xs
                    a2 = a2 + g2 * xs
                else:
                    b0 = b0 + g0 * xs
                    b1 = b1 + g1 * xs
                    b2 = b2 + g2 * xs
            return (a0, a1, a2, b0, b1, b2, c0, c1, c2)

        a0, a1, a2, b0, b1, b2, _, _, _ = lax.fori_loop(
            0, _D // _L, dloop, (zero,) * 6 + c_init)
        a0, a1, a2 = a0 + b0, a1 + b1, a2 + b2
        sq_acc = sq_acc + a0 * a0 + a1 * a1 + a2 * a2

    for s in (8, 4, 2, 1):
        red_v[...] = sq_acc
        sq_acc = sq_acc + plsc.load_gather(red_v, [(lane + s) % _L])

    out_v[...] = sq_acc
    pltpu.sync_copy(out_v, out_hbm.at[wid])


def _tc_body(w3_ref, der_ref, out_ref):
    s = []
    for bb in range(4):
        y = lax.dot_general(der_ref[bb], w3_ref[bb], (((1,), (1,)), ((), ())),
                            preferred_element_type=jnp.float32)
        s.append(jnp.sum(y * y, keepdims=True))
    out_ref[...] = jnp.stack(s)


def kernel(x, der_desc_wrt_coord):
    B, A, D, K = der_desc_wrt_coord.shape
    der3 = der_desc_wrt_coord.reshape(B, A, D * K)
    x_flat = x.reshape(B * D)

    sc = pl.kernel(
        _sc_body,
        out_type=jax.ShapeDtypeStruct((32, _L), jnp.float32),
        mesh=plsc.VectorSubcoreMesh(core_axis_name="c", subcore_axis_name="s"),
        compiler_params=pltpu.CompilerParams(needs_layout_passes=False),
        scratch_types=[
            pltpu.VMEM((_D,), jnp.float32),
            pltpu.VMEM((_CA, _R), jnp.float32),
            pltpu.VMEM((_CA, _R), jnp.float32),
            pltpu.VMEM((_L,), jnp.float32),
            pltpu.VMEM((_L,), jnp.float32),
            pltpu.SemaphoreType.DMA,
            pltpu.SemaphoreType.DMA,
        ],
    )
    out_sc = sc(x_flat, der3)

    j = jnp.arange(D * K, dtype=jnp.int32)
    sel = (j[None, :] % K) == jnp.arange(K, dtype=jnp.int32)[:, None]
    w3 = jnp.where(sel[None], x[:, None, j // K], 0.0)
    out_tc = pl.pallas_call(
        _tc_body,
        grid=((B - _NSC) // 4,),
        in_specs=[
            pl.BlockSpec((4, K, D * K), lambda b: (b + _NSC // 4, 0, 0)),
            pl.BlockSpec((4, A, D * K), lambda b: (b + _NSC // 4, 0, 0)),
        ],
        out_specs=pl.BlockSpec((4, 1, 1), lambda b: (b, 0, 0)),
        out_shape=jax.ShapeDtypeStruct((B - _NSC, 1, 1), jnp.float32),
    )(w3, der3)

    head = out_sc[:, 0].reshape(_NSC, 4).sum(axis=1)
    return jnp.concatenate([head, out_tc[:, 0, 0]])


# speedup vs baseline: 4.8288x; 1.0020x over previous
import jax
import jax.numpy as jnp
from jax import lax
from jax.experimental import pallas as pl
from jax.experimental.pallas import tpu as pltpu
from jax.experimental.pallas import tpu_sc as plsc

_L = 16
_CA = 16
_D = 512
_R = 3 * _D
_QA = 32
_NSC = 8


def _sc_body(x_hbm, der_hbm, out_hbm, x_v, der_v0, der_v1, out_v, red_v,
             sem0, sem1):
    wid = lax.axis_index("c") * 16 + lax.axis_index("s")
    q = wid // 4
    r = wid % 4
    lane = lax.broadcasted_iota(jnp.int32, (_L,), 0)

    pltpu.sync_copy(x_hbm.at[pl.ds(q * _D, _D)], x_v)

    bufs = (der_v0, der_v1)
    sems = (sem0, sem1)
    zero = jnp.zeros((_L,), jnp.float32)

    for u in range(2):
        pltpu.async_copy(
            der_hbm.at[q, pl.ds(r * _QA + u * _CA, _CA)], bufs[u], sems[u])

    c_init = (lane * 0, lane * 0 + 1, lane * 0 + 2)
    sq_acc = zero
    for u in range(2):
        buf, sem = bufs[u], sems[u]
        pltpu.make_async_copy(der_hbm.at[q, pl.ds(0, _CA)], buf, sem).wait()

        def dloop(i, carry, buf=buf):
            a0, a1, a2, b0, b1, b2, c0, c1, c2 = carry
            xv = x_v[pl.ds(i * _L, _L)]
            for m in range(_L):
                g0 = plsc.load_gather(buf, [lane, c0])
                g1 = plsc.load_gather(buf, [lane, c1])
                g2 = plsc.load_gather(buf, [lane, c2])
                c0, c1, c2 = c0 + 3, c1 + 3, c2 + 3
                xs = xv[m]
                if m % 2 == 0:
                    a0 = a0 + g0 * xs
                    a1 = a1 + g1 * xs
                    a2 = a2 + g2 * xs
                else:
                    b0 = b0 + g0 * xs
                    b1 = b1 + g1 * xs
                    b2 = b2 + g2 * xs
            return (a0, a1, a2, b0, b1, b2, c0, c1, c2)

        a0, a1, a2, b0, b1, b2, _, _, _ = lax.fori_loop(
            0, _D // _L, dloop, (zero,) * 6 + c_init)
        a0, a1, a2 = a0 + b0, a1 + b1, a2 + b2
        sq_acc = sq_acc + a0 * a0 + a1 * a1 + a2 * a2

    for s in (8, 4, 2, 1):
        red_v[...] = sq_acc
        sq_acc = sq_acc + plsc.load_gather(red_v, [(lane + s) % _L])

    out_v[...] = sq_acc
    pltpu.sync_copy(out_v, out_hbm.at[wid])


def _tc_body(w3_ref, der_ref, out_ref):
    s = []
    for bb in range(4):
        y = lax.dot_general(der_ref[bb], w3_ref[bb], (((1,), (1,)), ((), ())),
                            preferred_element_type=jnp.float32)
        s.append(jnp.sum(y * y, keepdims=True))
    out_ref[...] = jnp.stack(s)


def kernel(x, der_desc_wrt_coord):
    B, A, D, K = der_desc_wrt_coord.shape
    der3 = der_desc_wrt_coord.reshape(B, A, D * K)
    x_flat = x.reshape(B * D)

    sc = pl.kernel(
        _sc_body,
        out_type=jax.ShapeDtypeStruct((32, _L), jnp.float32),
        mesh=plsc.VectorSubcoreMesh(core_axis_name="c", subcore_axis_name="s"),
        compiler_params=pltpu.CompilerParams(needs_layout_passes=False),
        scratch_types=[
            pltpu.VMEM((_D,), jnp.float32),
            pltpu.VMEM((_CA, _R), jnp.float32),
            pltpu.VMEM((_CA, _R), jnp.float32),
            pltpu.VMEM((_L,), jnp.float32),
            pltpu.VMEM((_L,), jnp.float32),
            pltpu.SemaphoreType.DMA,
            pltpu.SemaphoreType.DMA,
        ],
    )
    out_sc = sc(x_flat, der3)

    j = jnp.arange(D * K, dtype=jnp.int32)
    sel = (j[None, :] % K) == jnp.arange(K, dtype=jnp.int32)[:, None]
    w3 = jnp.where(sel[None], x[:, None, j // K], 0.0)
    out_tc = pl.pallas_call(
        _tc_body,
        grid=((B - _NSC) // 4,),
        in_specs=[
            pl.BlockSpec((4, K, D * K), lambda b: (b + _NSC // 4, 0, 0)),
            pl.BlockSpec((4, A, D * K), lambda b: (b + _NSC // 4, 0, 0)),
        ],
        out_specs=pl.BlockSpec((4, 1, 1), lambda b: (b, 0, 0)),
        out_shape=jax.ShapeDtypeStruct((B - _NSC, 1, 1), jnp.float32),
    )(w3, der3)

    head = out_sc[:, 0].reshape(_NSC, 4).sum(axis=1)
    return jnp.concatenate([head, out_tc[:, 0, 0]])
